# Initial kernel scaffold; baseline (speedup 1.0000x reference)
import jax
import jax.numpy as jnp
from jax.experimental import pallas as pl


def kernel(xyz1, xyz2, points1, points2, W1, b1, g1, be1, W2, b2, g2, be2):
    raise NotImplementedError("write your pallas kernel here")


# speedup vs baseline: 21.9557x; 21.9557x over previous
import functools

import jax
import jax.numpy as jnp
from jax.experimental import pallas as pl


def _stage_a_body(xyz1_r, xyz2t_r, p2_r, p1_r, w1a_r, w1b_r, b1_r,
                  y1_r, st_r, *, M, NB):
    b = pl.program_id(0)
    nb = pl.program_id(1)
    q = xyz1_r[0]
    kt = xyz2t_r[0]
    d = None
    for c in range(3):
        qc = q[c:c + 1, :]
        kc = kt[:, c:c + 1]
        t = kc - qc
        d = t * t if d is None else d + t * t

    iota0 = jax.lax.broadcasted_iota(jnp.int32, (M, NB), 0)
    dm = d
    vals, idxs = [], []
    for k in range(3):
        mval = jnp.min(dm, axis=0, keepdims=True)
        midx = jnp.min(jnp.where(dm == mval, iota0, M), axis=0,
                       keepdims=True)
        vals.append(mval)
        idxs.append(midx)
        if k < 2:
            dm = jnp.where(iota0 == midx, jnp.float32(jnp.inf), dm)

    rs = [1.0 / (jnp.maximum(v, 1e-10) + 1e-8) for v in vals]
    rsum = rs[0] + rs[1] + rs[2]
    ws = [r / rsum for r in rs]

    oh = jnp.where(iota0 == idxs[0], ws[0], 0.0)
    oh = oh + jnp.where(iota0 == idxs[1], ws[1], 0.0)
    oh = oh + jnp.where(iota0 == idxs[2], ws[2], 0.0)

    interp = jnp.dot(p2_r[0], oh, preferred_element_type=jnp.float32)
    y = (jnp.dot(w1a_r[...], interp, preferred_element_type=jnp.float32)
         + jnp.dot(w1b_r[...], p1_r[0], preferred_element_type=jnp.float32)
         + b1_r[...])
    y1_r[0] = y

    s = jnp.sum(y, axis=1, keepdims=True)
    ss = jnp.sum(y * y, axis=1, keepdims=True)
    stv = jnp.concatenate([s, ss], axis=1)
    first = (b == 0) & (nb == 0)

    @pl.when(first)
    def _():
        st_r[...] = stv

    @pl.when(jnp.logical_not(first))
    def _():
        st_r[...] = st_r[...] + stv


def _stage_b_body(y1_r, w2_r, b2_r, a1_r, c1_r, y2_r, st_r):
    b = pl.program_id(0)
    nb = pl.program_id(1)
    h = jax.nn.relu(y1_r[0] * a1_r[...] + c1_r[...])
    y = jnp.dot(w2_r[...], h, preferred_element_type=jnp.float32) + b2_r[...]
    y2_r[0] = y
    s = jnp.sum(y, axis=1, keepdims=True)
    ss = jnp.sum(y * y, axis=1, keepdims=True)
    stv = jnp.concatenate([s, ss], axis=1)
    first = (b == 0) & (nb == 0)

    @pl.when(first)
    def _():
        st_r[...] = stv

    @pl.when(jnp.logical_not(first))
    def _():
        st_r[...] = st_r[...] + stv


def _stage_c_body(y2_r, a2_r, c2_r, out_r):
    out_r[0] = jax.nn.relu(y2_r[0] * a2_r[...] + c2_r[...])


def kernel(xyz1, xyz2, points1, points2, W1, b1, g1, be1, W2, b2, g2, be2):
    B, _, N = xyz1.shape
    M = xyz2.shape[2]
    C2 = points2.shape[1]
    C1 = points1.shape[1]
    H1 = W1.shape[0]
    H2 = W2.shape[0]
    NB = 512
    while N % NB:
        NB //= 2
    grid = (B, N // NB)
    cnt = float(B * N)

    xyz2t = jnp.transpose(xyz2, (0, 2, 1))
    W1a = W1[:, :C2]
    W1b = W1[:, C2:]
    b1c = b1[:, None]

    y1, st1 = pl.pallas_call(
        functools.partial(_stage_a_body, M=M, NB=NB),
        grid=grid,
        in_specs=[
            pl.BlockSpec((1, 3, NB), lambda b, n: (b, 0, n)),
            pl.BlockSpec((1, M, 3), lambda b, n: (b, 0, 0)),
            pl.BlockSpec((1, C2, M), lambda b, n: (b, 0, 0)),
            pl.BlockSpec((1, C1, NB), lambda b, n: (b, 0, n)),
            pl.BlockSpec((H1, C2), lambda b, n: (0, 0)),
            pl.BlockSpec((H1, C1), lambda b, n: (0, 0)),
            pl.BlockSpec((H1, 1), lambda b, n: (0, 0)),
        ],
        out_specs=[
            pl.BlockSpec((1, H1, NB), lambda b, n: (b, 0, n)),
            pl.BlockSpec((H1, 2), lambda b, n: (0, 0)),
        ],
        out_shape=[
            jax.ShapeDtypeStruct((B, H1, N), jnp.float32),
            jax.ShapeDtypeStruct((H1, 2), jnp.float32),
        ],
    )(xyz1, xyz2t, points2, points1, W1a, W1b, b1c)

    mu1 = st1[:, 0] / cnt
    var1 = st1[:, 1] / cnt - mu1 * mu1
    a1 = g1 / jnp.sqrt(var1 + 1e-5)
    c1 = be1 - mu1 * a1

    y2, st2 = pl.pallas_call(
        _stage_b_body,
        grid=grid,
        in_specs=[
            pl.BlockSpec((1, H1, NB), lambda b, n: (b, 0, n)),
            pl.BlockSpec((H2, H1), lambda b, n: (0, 0)),
            pl.BlockSpec((H2, 1), lambda b, n: (0, 0)),
            pl.BlockSpec((H1, 1), lambda b, n: (0, 0)),
            pl.BlockSpec((H1, 1), lambda b, n: (0, 0)),
        ],
        out_specs=[
            pl.BlockSpec((1, H2, NB), lambda b, n: (b, 0, n)),
            pl.BlockSpec((H2, 2), lambda b, n: (0, 0)),
        ],
        out_shape=[
            jax.ShapeDtypeStruct((B, H2, N), jnp.float32),
            jax.ShapeDtypeStruct((H2, 2), jnp.float32),
        ],
    )(y1, W2, b2[:, None], a1[:, None], c1[:, None])

    mu2 = st2[:, 0] / cnt
    var2 = st2[:, 1] / cnt - mu2 * mu2
    a2 = g2 / jnp.sqrt(var2 + 1e-5)
    c2 = be2 - mu2 * a2

    out = pl.pallas_call(
        _stage_c_body,
        grid=grid,
        in_specs=[
            pl.BlockSpec((1, H2, NB), lambda b, n: (b, 0, n)),
            pl.BlockSpec((H2, 1), lambda b, n: (0, 0)),
            pl.BlockSpec((H2, 1), lambda b, n: (0, 0)),
        ],
        out_specs=pl.BlockSpec((1, H2, NB), lambda b, n: (b, 0, n)),
        out_shape=jax.ShapeDtypeStruct((B, H2, N), jnp.float32),
    )(y2, a2[:, None], c2[:, None])

    return out
